# Initial kernel scaffold; baseline (speedup 1.0000x reference)
import functools

import jax
import jax.numpy as jnp
from jax import lax
from jax.experimental import pallas as pl
from jax.experimental.pallas import tpu as pltpu
from jax.experimental.pallas import tpu_sc as plsc

N = 10000
D = 128
E = 320000
L = 16
NC = 2
NS = 16
EP = 323584
EROWS = EP // 128
P1_ROWS = EROWS // NS
P2_ROWS = EROWS // (NC * NS)
NP = 10008
NTC = 10240

_mesh = plsc.VectorSubcoreMesh(
    core_axis_name="c", subcore_axis_name="s", num_cores=NC, num_subcores=NS)


def _stage1(xp, W, a_s2, a_d2):
    def body(x_ref, w_ref, as_ref, ad_ref, h_ref, s_ref, d_ref):
        h = lax.dot_general(x_ref[...], w_ref[...],
                            (((1,), (1,)), ((), ())),
                            preferred_element_type=jnp.float32)
        h_ref[...] = h
        s_ref[...] = jnp.sum(h * as_ref[...], axis=1, keepdims=True)
        d_ref[...] = jnp.sum(h * ad_ref[...], axis=1, keepdims=True)

    return pl.pallas_call(
        body,
        grid=(NTC // 1024,),
        in_specs=[
            pl.BlockSpec((1024, D), lambda i: (i, 0)),
            pl.BlockSpec((D, D), lambda i: (0, 0)),
            pl.BlockSpec((1, D), lambda i: (0, 0)),
            pl.BlockSpec((1, D), lambda i: (0, 0)),
        ],
        out_specs=[
            pl.BlockSpec((1024, D), lambda i: (i, 0)),
            pl.BlockSpec((1024, 1), lambda i: (i, 0)),
            pl.BlockSpec((1024, 1), lambda i: (i, 0)),
        ],
        out_shape=[
            jax.ShapeDtypeStruct((NTC, D), jnp.float32),
            jax.ShapeDtypeStruct((NTC, 1), jnp.float32),
            jax.ShapeDtypeStruct((NTC, 1), jnp.float32),
        ],
    )(xp, W, a_s2, a_d2)


def _edge_logits(sv, dv):
    e = sv + dv
    e = jnp.where(e > 0, e, 0.2 * e)
    e = jnp.minimum(jnp.maximum(e, -10.0), 10.0)
    return jnp.exp(e)


@functools.partial(
    pl.kernel,
    out_type=jax.ShapeDtypeStruct((NC, N, D), jnp.float32),
    mesh=_mesh,
    scratch_types=[
        pltpu.VMEM_SHARED((NP,), jnp.float32),
        pltpu.VMEM_SHARED((NP, D), jnp.float32),
        pltpu.VMEM((NP,), jnp.float32),
        pltpu.VMEM((NP,), jnp.float32),
        pltpu.VMEM((NP,), jnp.float32),
        pltpu.VMEM((2, 128), jnp.int32),
        pltpu.VMEM((2, 128), jnp.int32),
        pltpu.VMEM((2, 128), jnp.float32),
        pltpu.VMEM((1, 128), jnp.int32),
        pltpu.VMEM((1, 128), jnp.int32),
        pltpu.VMEM((128,), jnp.float32),
        pltpu.VMEM((158, D), jnp.float32),
        pltpu.VMEM((640,), jnp.float32),
        pltpu.VMEM((128, D), jnp.float32),
        pltpu.SemaphoreType.DMA,
    ],
)
def _sc_stage(src2d, dst2d, s1, d1, h, outp,
              denom_sh, out_sh, s_loc, d_loc, denom_loc,
              src_i, dst_i, vals, src_c, dst_c, alpha_b,
              zrow, zden, rows_v, sem):
    c = lax.axis_index("c")
    sid = lax.axis_index("s")
    zero16 = jnp.zeros((L,), jnp.float32)

    @pl.loop(0, 158)
    def _zr(r):
        for q in range(8):
            zrow[r, pl.ds(q * L, L)] = zero16

    @pl.loop(0, 40)
    def _zd(i):
        zden[pl.ds(i * L, L)] = zero16

    start_o = jnp.minimum(sid * 632, NP - 632)
    for k in range(4):
        pltpu.sync_copy(zrow, out_sh.at[pl.ds(start_o + k * 158, 158)])
    start_d = jnp.minimum(sid * 640, NP - 640)
    pltpu.sync_copy(zden, denom_sh.at[pl.ds(start_d, 640)])
    pltpu.sync_copy(s1.at[pl.ds(0, NP)], s_loc)
    pltpu.sync_copy(d1.at[pl.ds(0, NP)], d_loc)
    plsc.subcore_barrier()

    p1base = sid * P1_ROWS

    @pl.loop(0, P1_ROWS // 2)
    def _p1(t):
        r0 = p1base + t * 2
        pltpu.sync_copy(src2d.at[pl.ds(r0, 2)], src_i)
        pltpu.sync_copy(dst2d.at[pl.ds(r0, 2)], dst_i)
        for j in range(2):
            for g in range(8):
                sidx = src_i[j, pl.ds(g * L, L)]
                didx = dst_i[j, pl.ds(g * L, L)]
                sv = plsc.load_gather(s_loc, [sidx])
                dv = plsc.load_gather(d_loc, [didx])
                vals[j, pl.ds(g * L, L)] = _edge_logits(sv, dv)
            pltpu.sync_copy(vals.at[j], denom_sh.at[dst_i.at[j]], add=True)

    plsc.subcore_barrier()
    pltpu.sync_copy(denom_sh, denom_loc)

    wid = sid * NC + c
    p2base = wid * P2_ROWS

    @pl.loop(0, P2_ROWS)
    def _p2(t):
        r = p2base + t
        pltpu.sync_copy(src2d.at[pl.ds(r, 1)], src_c)
        pltpu.sync_copy(dst2d.at[pl.ds(r, 1)], dst_c)
        pltpu.async_copy(h.at[src_c.at[0]], rows_v, sem).wait()
        for g in range(8):
            sidx = src_c[0, pl.ds(g * L, L)]
            didx = dst_c[0, pl.ds(g * L, L)]
            sv = plsc.load_gather(s_loc, [sidx])
            dv = plsc.load_gather(d_loc, [didx])
            anum = _edge_logits(sv, dv)
            den = plsc.load_gather(denom_loc, [didx])
            alpha_b[pl.ds(g * L, L)] = anum / (den + 1e-9)

        @pl.loop(0, 128)
        def _scale(e2):
            a = alpha_b[e2]
            for q in range(8):
                rows_v[e2, pl.ds(q * L, L)] = rows_v[e2, pl.ds(q * L, L)] * a

        pltpu.sync_copy(rows_v, out_sh.at[dst_c.at[0]], add=True)

    plsc.subcore_barrier()

    start_w = jnp.minimum(sid * 632, N - 632)
    pltpu.sync_copy(out_sh.at[pl.ds(start_w, 632)],
                    outp.at[c, pl.ds(start_w, 632)])


def _stage3(p0, p1):
    def body(a_ref, b_ref, o_ref):
        o_ref[...] = a_ref[...] + b_ref[...]

    return pl.pallas_call(
        body,
        grid=(10,),
        in_specs=[pl.BlockSpec((1000, D), lambda i: (i, 0)),
                  pl.BlockSpec((1000, D), lambda i: (i, 0))],
        out_specs=pl.BlockSpec((1000, D), lambda i: (i, 0)),
        out_shape=jax.ShapeDtypeStruct((N, D), jnp.float32),
    )(p0, p1)


def kernel(x, edge_index, W, a_src, a_dst):
    xp = jnp.pad(x, ((0, NTC - N), (0, 0)))
    h, s2, d2 = _stage1(xp, W, a_src.reshape(1, D), a_dst.reshape(1, D))
    s1 = s2.reshape(NTC)
    d1 = d2.reshape(NTC)
    src_p = jnp.concatenate(
        [edge_index[0], jnp.zeros((EP - E,), jnp.int32)]).reshape(EROWS, 128)
    dst_p = jnp.concatenate(
        [edge_index[1], jnp.full((EP - E,), N, jnp.int32)]).reshape(EROWS, 128)
    outp = _sc_stage(src_p, dst_p, s1, d1, h)
    return _stage3(outp[0], outp[1])


# speedup vs baseline: 9.3213x; 9.3213x over previous
import functools

import jax
import jax.numpy as jnp
from jax import lax
from jax.experimental import pallas as pl
from jax.experimental.pallas import tpu as pltpu
from jax.experimental.pallas import tpu_sc as plsc

N = 10000
D = 128
E = 320000
L = 16
NC = 2
NS = 16
CPT = 80
EP = NC * NS * CPT * 128
EROWS = EP // 128
NP = 10008
NTC = 10240

_mesh = plsc.VectorSubcoreMesh(
    core_axis_name="c", subcore_axis_name="s", num_cores=NC, num_subcores=NS)


def _stage1(xp, W, a_s2, a_d2):
    def body(x_ref, w_ref, as_ref, ad_ref, h_ref, s_ref, d_ref):
        h = lax.dot_general(x_ref[...], w_ref[...],
                            (((1,), (1,)), ((), ())),
                            preferred_element_type=jnp.float32)
        h_ref[...] = h
        s_ref[...] = jnp.sum(h * as_ref[...], axis=1, keepdims=True)
        d_ref[...] = jnp.sum(h * ad_ref[...], axis=1, keepdims=True)

    return pl.pallas_call(
        body,
        grid=(NTC // 1024,),
        in_specs=[
            pl.BlockSpec((1024, D), lambda i: (i, 0)),
            pl.BlockSpec((D, D), lambda i: (0, 0)),
            pl.BlockSpec((1, D), lambda i: (0, 0)),
            pl.BlockSpec((1, D), lambda i: (0, 0)),
        ],
        out_specs=[
            pl.BlockSpec((1024, D), lambda i: (i, 0)),
            pl.BlockSpec((1024, 1), lambda i: (i, 0)),
            pl.BlockSpec((1024, 1), lambda i: (i, 0)),
        ],
        out_shape=[
            jax.ShapeDtypeStruct((NTC, D), jnp.float32),
            jax.ShapeDtypeStruct((NTC, 1), jnp.float32),
            jax.ShapeDtypeStruct((NTC, 1), jnp.float32),
        ],
    )(xp, W, a_s2, a_d2)


def _edge_logits(sv, dv):
    e = sv + dv
    e = jnp.where(e > 0, e, 0.2 * e)
    e = jnp.minimum(jnp.maximum(e, -10.0), 10.0)
    return jnp.exp(e)


@functools.partial(
    pl.kernel,
    out_type=[jax.ShapeDtypeStruct((NC, N, D), jnp.float32),
              jax.ShapeDtypeStruct((NP,), jnp.float32),
              jax.ShapeDtypeStruct((NP,), jnp.float32)],
    mesh=_mesh,
    compiler_params=pltpu.CompilerParams(needs_layout_passes=False),
    scratch_types=[
        pltpu.VMEM_SHARED((NP,), jnp.float32),
        pltpu.VMEM_SHARED((NP, D), jnp.float32),
        pltpu.VMEM((1, 128), jnp.int32),
        pltpu.VMEM((1, 128), jnp.int32),
        pltpu.VMEM((1, 128), jnp.int32),
        pltpu.VMEM((1, 128), jnp.int32),
        pltpu.VMEM((1, 128), jnp.float32),
        pltpu.VMEM((1, 128), jnp.float32),
        pltpu.VMEM((1, 128), jnp.float32),
        pltpu.VMEM((1, 128), jnp.float32),
        pltpu.VMEM((1, 128), jnp.float32),
        pltpu.VMEM((640,), jnp.float32),
        pltpu.VMEM((128, D), jnp.float32),
        pltpu.VMEM((128, D), jnp.float32),
        pltpu.SemaphoreType.DMA,
        pltpu.SemaphoreType.DMA,
    ],
)
def _sc_stage(src2d, dst2d, s1, d1, h, outp, denp0, denp1,
              denom_sh, out_sh, src0, src1, dst0, dst1,
              sv0, sv1, dv0, dv1, vals, zden,
              rows0, rows1, semg0, semg1):
    c = lax.axis_index("c")
    sid = lax.axis_index("s")
    srcb = (src0, src1)
    dstb = (dst0, dst1)
    svb = (sv0, sv1)
    dvb = (dv0, dv1)
    rows = (rows0, rows1)
    semg = (semg0, semg1)
    zero16 = jnp.zeros((L,), jnp.float32)

    @pl.loop(0, 128)
    def _zr(r):
        for q in range(8):
            rows0[r, pl.ds(q * L, L)] = zero16

    @pl.loop(0, 40)
    def _zd(i):
        zden[pl.ds(i * L, L)] = zero16

    start_o = jnp.minimum(sid * 632, NP - 632)
    for k in range(4):
        pltpu.sync_copy(rows0, out_sh.at[pl.ds(start_o + k * 128, 128)])
    pltpu.sync_copy(rows0.at[pl.ds(0, 120)],
                    out_sh.at[pl.ds(start_o + 512, 120)])
    start_d = jnp.minimum(sid * 640, NP - 640)
    pltpu.sync_copy(zden, denom_sh.at[pl.ds(start_d, 640)])
    plsc.subcore_barrier()

    wid = sid * NC + c
    base = wid * CPT

    def load_idx(b, r):
        pltpu.sync_copy(src2d.at[pl.ds(r, 1)], srcb[b])
        pltpu.sync_copy(dst2d.at[pl.ds(r, 1)], dstb[b])

    def issue(b):
        pltpu.async_copy(s1.at[srcb[b].at[0]], svb[b].at[0], semg[b])
        pltpu.async_copy(d1.at[dstb[b].at[0]], dvb[b].at[0], semg[b])
        pltpu.async_copy(h.at[srcb[b].at[0]], rows[b], semg[b])

    def wait_g(b):
        pltpu.make_async_copy(
            s1.at[srcb[b].at[0]], svb[b].at[0], semg[b]).wait()
        pltpu.make_async_copy(
            d1.at[dstb[b].at[0]], dvb[b].at[0], semg[b]).wait()
        pltpu.make_async_copy(h.at[srcb[b].at[0]], rows[b], semg[b]).wait()

    def process(b):
        for g in range(8):
            sv = svb[b][0, pl.ds(g * L, L)]
            dv = dvb[b][0, pl.ds(g * L, L)]
            vals[0, pl.ds(g * L, L)] = _edge_logits(sv, dv)
        pltpu.sync_copy(vals.at[0], denom_sh.at[dstb[b].at[0]], add=True)
        for g2 in range(8):
            a16 = vals[0, pl.ds(g2 * L, L)]
            for lane in range(L):
                e2 = g2 * L + lane
                a = a16[lane]
                for q in range(8):
                    rows[b][e2, pl.ds(q * L, L)] = (
                        rows[b][e2, pl.ds(q * L, L)] * a)
        pltpu.sync_copy(rows[b], out_sh.at[dstb[b].at[0]], add=True)

    load_idx(0, base)
    issue(0)

    @pl.loop(0, CPT // 2)
    def _main(t2):
        r0 = base + 2 * t2
        load_idx(1, r0 + 1)
        issue(1)
        wait_g(0)
        process(0)

        @pl.when(t2 < CPT // 2 - 1)
        def _pref():
            load_idx(0, r0 + 2)
            issue(0)

        wait_g(1)
        process(1)

    plsc.subcore_barrier()

    start_w = jnp.minimum(sid * 632, N - 632)
    pltpu.sync_copy(out_sh.at[pl.ds(start_w, 632)],
                    outp.at[c, pl.ds(start_w, 632)])

    pltpu.sync_copy(denom_sh.at[pl.ds(start_d, 640)], zden)

    @pl.when(c == 0)
    def _wd0():
        pltpu.sync_copy(zden, denp0.at[pl.ds(start_d, 640)])

    @pl.when(c == 1)
    def _wd1():
        pltpu.sync_copy(zden, denp1.at[pl.ds(start_d, 640)])


def _stage3(p0, p1, d0, d1):
    def body(a_ref, b_ref, da_ref, db_ref, o_ref):
        inv = 1.0 / (da_ref[...] + db_ref[...] + 1e-9)
        o_ref[...] = (a_ref[...] + b_ref[...]) * inv

    return pl.pallas_call(
        body,
        grid=(10,),
        in_specs=[pl.BlockSpec((1000, D), lambda i: (i, 0)),
                  pl.BlockSpec((1000, D), lambda i: (i, 0)),
                  pl.BlockSpec((1000, 1), lambda i: (i, 0)),
                  pl.BlockSpec((1000, 1), lambda i: (i, 0))],
        out_specs=pl.BlockSpec((1000, D), lambda i: (i, 0)),
        out_shape=jax.ShapeDtypeStruct((N, D), jnp.float32),
    )(p0, p1, d0, d1)


def kernel(x, edge_index, W, a_src, a_dst):
    xp = jnp.pad(x, ((0, NTC - N), (0, 0)))
    h, s2, d2 = _stage1(xp, W, a_src.reshape(1, D), a_dst.reshape(1, D))
    s1 = s2.reshape(NTC)
    d1 = d2.reshape(NTC)
    src_p = jnp.concatenate(
        [edge_index[0], jnp.zeros((EP - E,), jnp.int32)]).reshape(EROWS, 128)
    dst_p = jnp.concatenate(
        [edge_index[1], jnp.full((EP - E,), N, jnp.int32)]).reshape(EROWS, 128)
    outp, denp0, denp1 = _sc_stage(src_p, dst_p, s1, d1, h)
    d0 = denp0[:N].reshape(N, 1)
    d1_ = denp1[:N].reshape(N, 1)
    return _stage3(outp[0], outp[1], d0, d1_)


# speedup vs baseline: 13.4455x; 1.4424x over previous
import functools

import jax
import jax.numpy as jnp
from jax import lax
from jax.experimental import pallas as pl
from jax.experimental.pallas import tpu as pltpu
from jax.experimental.pallas import tpu_sc as plsc

N = 10000
D = 128
E = 320000
L = 16
NC = 2
NS = 16
CPT = 80
EP = NC * NS * CPT * 128
EROWS = EP // 128
NP = 10008
NTC = 10240

_mesh = plsc.VectorSubcoreMesh(
    core_axis_name="c", subcore_axis_name="s", num_cores=NC, num_subcores=NS)


def _stage1(xp, W, a_s2, a_d2):
    def body(x_ref, w_ref, as_ref, ad_ref, h_ref, s_ref, d_ref):
        h = lax.dot_general(x_ref[...], w_ref[...],
                            (((1,), (1,)), ((), ())),
                            preferred_element_type=jnp.float32)
        h_ref[...] = h
        s_ref[...] = jnp.sum(h * as_ref[...], axis=1, keepdims=True)
        d_ref[...] = jnp.sum(h * ad_ref[...], axis=1, keepdims=True)

    return pl.pallas_call(
        body,
        grid=(NTC // 1024,),
        in_specs=[
            pl.BlockSpec((1024, D), lambda i: (i, 0)),
            pl.BlockSpec((D, D), lambda i: (0, 0)),
            pl.BlockSpec((1, D), lambda i: (0, 0)),
            pl.BlockSpec((1, D), lambda i: (0, 0)),
        ],
        out_specs=[
            pl.BlockSpec((1024, D), lambda i: (i, 0)),
            pl.BlockSpec((1024, 1), lambda i: (i, 0)),
            pl.BlockSpec((1024, 1), lambda i: (i, 0)),
        ],
        out_shape=[
            jax.ShapeDtypeStruct((NTC, D), jnp.float32),
            jax.ShapeDtypeStruct((NTC, 1), jnp.float32),
            jax.ShapeDtypeStruct((NTC, 1), jnp.float32),
        ],
    )(xp, W, a_s2, a_d2)


def _edge_logits(sv, dv):
    e = sv + dv
    e = jnp.where(e > 0, e, 0.2 * e)
    e = jnp.minimum(jnp.maximum(e, -10.0), 10.0)
    return jnp.exp(e)


@functools.partial(
    pl.kernel,
    out_type=[jax.ShapeDtypeStruct((NC, N, D), jnp.float32),
              jax.ShapeDtypeStruct((NP,), jnp.float32),
              jax.ShapeDtypeStruct((NP,), jnp.float32)],
    mesh=_mesh,
    compiler_params=pltpu.CompilerParams(needs_layout_passes=False),
    scratch_types=[
        pltpu.VMEM_SHARED((NP,), jnp.float32),
        pltpu.VMEM_SHARED((NP, D), jnp.float32),
        pltpu.VMEM((1, 128), jnp.int32),
        pltpu.VMEM((1, 128), jnp.int32),
        pltpu.VMEM((1, 128), jnp.int32),
        pltpu.VMEM((1, 128), jnp.int32),
        pltpu.VMEM((1, 128), jnp.float32),
        pltpu.VMEM((1, 128), jnp.float32),
        pltpu.VMEM((1, 128), jnp.float32),
        pltpu.VMEM((1, 128), jnp.float32),
        pltpu.VMEM((1, 128), jnp.float32),
        pltpu.VMEM((640,), jnp.float32),
        pltpu.VMEM((128, D), jnp.float32),
        pltpu.VMEM((128, D), jnp.float32),
        pltpu.SemaphoreType.DMA,
        pltpu.SemaphoreType.DMA,
    ],
)
def _sc_stage(src2d, dst2d, s1, d1, h, outp, denp0, denp1,
              denom_sh, out_sh, src0, src1, dst0, dst1,
              sv0, sv1, dv0, dv1, vals, zden,
              rows0, rows1, semg0, semg1):
    c = lax.axis_index("c")
    sid = lax.axis_index("s")
    srcb = (src0, src1)
    dstb = (dst0, dst1)
    svb = (sv0, sv1)
    dvb = (dv0, dv1)
    rows = (rows0, rows1)
    semg = (semg0, semg1)
    zero16 = jnp.zeros((L,), jnp.float32)

    @pl.loop(0, 128)
    def _zr(r):
        for q in range(8):
            rows0[r, pl.ds(q * L, L)] = zero16

    @pl.loop(0, 40)
    def _zd(i):
        zden[pl.ds(i * L, L)] = zero16

    start_o = jnp.minimum(sid * 632, NP - 632)
    for k in range(4):
        pltpu.sync_copy(rows0, out_sh.at[pl.ds(start_o + k * 128, 128)])
    pltpu.sync_copy(rows0.at[pl.ds(0, 120)],
                    out_sh.at[pl.ds(start_o + 512, 120)])
    start_d = jnp.minimum(sid * 640, NP - 640)
    pltpu.sync_copy(zden, denom_sh.at[pl.ds(start_d, 640)])
    plsc.subcore_barrier()

    wid = sid * NC + c
    base = wid * CPT
    nv2 = jnp.maximum(0, jnp.minimum(CPT, 2500 - base)) // 2

    def load_idx(b, r):
        pltpu.sync_copy(src2d.at[pl.ds(r, 1)], srcb[b])
        pltpu.sync_copy(dst2d.at[pl.ds(r, 1)], dstb[b])

    def issue(b):
        pltpu.async_copy(s1.at[srcb[b].at[0]], svb[b].at[0], semg[b])
        pltpu.async_copy(d1.at[dstb[b].at[0]], dvb[b].at[0], semg[b])
        pltpu.async_copy(h.at[srcb[b].at[0]], rows[b], semg[b])

    def wait_g(b):
        pltpu.make_async_copy(
            s1.at[srcb[b].at[0]], svb[b].at[0], semg[b]).wait()
        pltpu.make_async_copy(
            d1.at[dstb[b].at[0]], dvb[b].at[0], semg[b]).wait()
        pltpu.make_async_copy(h.at[srcb[b].at[0]], rows[b], semg[b]).wait()

    def process(b):
        for g in range(8):
            sv = svb[b][0, pl.ds(g * L, L)]
            dv = dvb[b][0, pl.ds(g * L, L)]
            vals[0, pl.ds(g * L, L)] = _edge_logits(sv, dv)
        pltpu.sync_copy(vals.at[0], denom_sh.at[dstb[b].at[0]], add=True)
        for g2 in range(8):
            a16 = vals[0, pl.ds(g2 * L, L)]
            for lane in range(L):
                e2 = g2 * L + lane
                a = a16[lane]
                for q in range(8):
                    rows[b][e2, pl.ds(q * L, L)] = (
                        rows[b][e2, pl.ds(q * L, L)] * a)
        pltpu.sync_copy(rows[b], out_sh.at[dstb[b].at[0]], add=True)

    load_idx(0, base)
    issue(0)

    @pl.loop(0, nv2)
    def _main(t2):
        r0 = base + 2 * t2
        load_idx(1, r0 + 1)
        issue(1)
        wait_g(0)
        process(0)

        @pl.when(t2 < nv2 - 1)
        def _pref():
            load_idx(0, r0 + 2)
            issue(0)

        wait_g(1)
        process(1)

    plsc.subcore_barrier()

    start_w = jnp.minimum(sid * 632, N - 632)
    pltpu.sync_copy(out_sh.at[pl.ds(start_w, 632)],
                    outp.at[c, pl.ds(start_w, 632)])

    pltpu.sync_copy(denom_sh.at[pl.ds(start_d, 640)], zden)

    @pl.when(c == 0)
    def _wd0():
        pltpu.sync_copy(zden, denp0.at[pl.ds(start_d, 640)])

    @pl.when(c == 1)
    def _wd1():
        pltpu.sync_copy(zden, denp1.at[pl.ds(start_d, 640)])


def _stage3(p0, p1, d0, d1):
    def body(a_ref, b_ref, da_ref, db_ref, o_ref):
        inv = 1.0 / (da_ref[...] + db_ref[...] + 1e-9)
        o_ref[...] = (a_ref[...] + b_ref[...]) * inv

    return pl.pallas_call(
        body,
        grid=(10,),
        in_specs=[pl.BlockSpec((1000, D), lambda i: (i, 0)),
                  pl.BlockSpec((1000, D), lambda i: (i, 0)),
                  pl.BlockSpec((1000, 1), lambda i: (i, 0)),
                  pl.BlockSpec((1000, 1), lambda i: (i, 0))],
        out_specs=pl.BlockSpec((1000, D), lambda i: (i, 0)),
        out_shape=jax.ShapeDtypeStruct((N, D), jnp.float32),
    )(p0, p1, d0, d1)


def kernel(x, edge_index, W, a_src, a_dst):
    xp = jnp.pad(x, ((0, NTC - N), (0, 0)))
    h, s2, d2 = _stage1(xp, W, a_src.reshape(1, D), a_dst.reshape(1, D))
    s1 = s2.reshape(NTC)
    d1 = d2.reshape(NTC)
    src_p = jnp.concatenate(
        [edge_index[0], jnp.zeros((EP - E,), jnp.int32)]).reshape(EROWS, 128)
    dst_p = jnp.concatenate(
        [edge_index[1], jnp.full((EP - E,), N, jnp.int32)]).reshape(EROWS, 128)
    outp, denp0, denp1 = _sc_stage(src_p, dst_p, s1, d1, h)
    d0 = denp0[:N].reshape(N, 1)
    d1_ = denp1[:N].reshape(N, 1)
    return _stage3(outp[0], outp[1], d0, d1_)
